# baseline (speedup 1.0000x reference)
import functools

import jax
import jax.numpy as jnp
from jax import lax
from jax.experimental import pallas as pl
from jax.experimental.pallas import tpu as pltpu, tpu_sc as plsc

NUM_CORES = 2
NUM_SUBCORES = 16
NUM_WORKERS = NUM_CORES * NUM_SUBCORES
LANE = 16
ROW_W = 128


def _build_sc_call(num_entities, num_relations, num_edges):
    assert num_edges % ROW_W == 0
    rows_total = num_edges // ROW_W
    win_rows = 16
    win_edges = win_rows * ROW_W
    full_blocks = rows_total // win_rows
    tail_rows = rows_total - full_blocks * win_rows
    per = full_blocks // NUM_WORKERS
    rem = full_blocks % NUM_WORKERS
    assert per % 4 == 0
    n_groups = per // 4
    slice_len = ((num_entities + NUM_SUBCORES - 1) // NUM_SUBCORES + LANE - 1) // LANE * LANE
    ypad = slice_len * NUM_SUBCORES

    mesh = plsc.VectorSubcoreMesh(core_axis_name="c", subcore_axis_name="s")

    @functools.partial(
        pl.kernel,
        mesh=mesh,
        out_type=jax.ShapeDtypeStruct((NUM_CORES * ypad,), jnp.float32),
        compiler_params=pltpu.CompilerParams(needs_layout_passes=False),
        scratch_types=[
            pltpu.VMEM((num_entities,), jnp.float32),
            pltpu.VMEM((ROW_W,), jnp.float32),
            pltpu.VMEM((slice_len,), jnp.float32),
            [pltpu.VMEM((2, win_edges), jnp.int32) for _ in range(4)],
            [pltpu.VMEM((win_edges,), jnp.int32) for _ in range(4)],
            [pltpu.VMEM((win_rows, ROW_W), jnp.float32) for _ in range(4)],
            pltpu.VMEM_SHARED((ypad,), jnp.float32),
            [pltpu.SemaphoreType.DMA for _ in range(4)],
            [pltpu.SemaphoreType.DMA for _ in range(4)],
            pltpu.SemaphoreType.DMA,
        ],
    )
    def sc_call(ind_hbm, rel_hbm, e_hbm, p_hbm, out_hbm,
                e_t, p_t, tmp, pair_b, rel_b, con_b, acc,
                sem_in, sem_sc, sem_tab):
        cid = lax.axis_index("c")
        sid = lax.axis_index("s")
        wid = cid * NUM_SUBCORES + sid

        start_block = wid * per + jnp.minimum(wid, rem)

        def in_start(w, r):
            eb = (start_block + w) * win_edges
            pltpu.async_copy(ind_hbm.at[pl.ds(0, 2), pl.ds(eb, win_edges)],
                             pair_b[r], sem_in[r])
            pltpu.async_copy(rel_hbm.at[pl.ds(eb, win_edges)], rel_b[r],
                             sem_in[r])

        def in_wait(w, r):
            eb = (start_block + w) * win_edges
            pltpu.make_async_copy(
                ind_hbm.at[pl.ds(0, 2), pl.ds(eb, win_edges)], pair_b[r],
                sem_in[r]).wait()
            pltpu.make_async_copy(rel_hbm.at[pl.ds(eb, win_edges)], rel_b[r],
                                  sem_in[r]).wait()

        def compute(r, nrows=win_rows):
            for g in range(nrows * (ROW_W // LANE)):
                off = g * LANE
                rows = pair_b[r][0, pl.ds(off, LANE)]
                rels = rel_b[r][pl.ds(off, LANE)]
                ev = plsc.load_gather(e_t, [rows])
                pv = plsc.load_gather(p_t, [rels])
                con_b[r][g // (ROW_W // LANE),
                         pl.ds((g % (ROW_W // LANE)) * LANE, LANE)] = ev * pv

        def scat_fire(r, nrows=win_rows):
            for j in range(nrows):
                pltpu.async_copy(con_b[r].at[j],
                                 acc.at[pair_b[r].at[1, pl.ds(j * ROW_W, ROW_W)]],
                                 sem_sc[r], add=True)

        def scat_drain(r, nrows=win_rows):
            for j in range(nrows):
                pltpu.make_async_copy(
                    con_b[r].at[j],
                    acc.at[pair_b[r].at[1, pl.ds(j * ROW_W, ROW_W)]],
                    sem_sc[r]).wait()

        pltpu.async_copy(e_hbm, e_t, sem_tab)
        pltpu.async_copy(p_hbm, p_t.at[pl.ds(0, num_relations)], sem_tab)
        in_start(0, 0)
        in_start(1, 1)

        def _zero(i, carry):
            tmp[pl.ds(i * LANE, LANE)] = jnp.zeros((LANE,), jnp.float32)
            return carry
        lax.fori_loop(0, slice_len // LANE, _zero, 0)
        pltpu.sync_copy(tmp, acc.at[pl.ds(sid * slice_len, slice_len)])
        pltpu.make_async_copy(e_hbm, e_t, sem_tab).wait()
        pltpu.make_async_copy(p_hbm, p_t.at[pl.ds(0, num_relations)],
                              sem_tab).wait()
        plsc.subcore_barrier()

        def _group(g, carry):
            for k in range(4):
                w = g * 4 + k

                @pl.when(w >= 2)
                def _():
                    scat_drain((k + 2) % 4)

                @pl.when(w + 2 < per)
                def _():
                    in_start(w + 2, (k + 2) % 4)

                in_wait(w, k)
                compute(k)
                scat_fire(k)
            return carry
        lax.fori_loop(0, n_groups, _group, 0)

        scat_drain((per - 2) % 4)
        scat_drain((per - 1) % 4)

        def tail_window(base_row, nrows):
            eb = base_row * ROW_W
            ne = nrows * ROW_W
            pltpu.sync_copy(ind_hbm.at[pl.ds(0, 2), pl.ds(eb, ne)],
                            pair_b[0].at[pl.ds(0, 2), pl.ds(0, ne)])
            pltpu.sync_copy(rel_hbm.at[pl.ds(eb, ne)],
                            rel_b[0].at[pl.ds(0, ne)])
            compute(0, nrows)
            scat_fire(0, nrows)
            scat_drain(0, nrows)

        @pl.when(wid < rem)
        def _():
            tail_window((start_block + per) * win_rows, win_rows)

        if tail_rows:
            @pl.when(wid == NUM_WORKERS - 1)
            def _():
                tail_window(full_blocks * win_rows, tail_rows)

        plsc.subcore_barrier()
        pltpu.sync_copy(acc.at[pl.ds(sid * slice_len, slice_len)], tmp)
        pltpu.sync_copy(tmp, out_hbm.at[pl.ds(cid * ypad + sid * slice_len,
                                              slice_len)])

    return sc_call


def kernel(indices, e_scores, p_scores, relation_mask):
    num_entities = e_scores.shape[0]
    num_relations = p_scores.shape[0]
    num_edges = relation_mask.shape[0]

    e_flat = e_scores.reshape(num_entities)

    sc_call = _build_sc_call(num_entities, num_relations, num_edges)
    flat = sc_call(indices, relation_mask, e_flat, p_scores)
    partials = flat.reshape(NUM_CORES, flat.shape[0] // NUM_CORES)
    return partials[0, :num_entities] + partials[1, :num_entities]


# speedup vs baseline: 1.1074x; 1.1074x over previous
import functools

import jax
import jax.numpy as jnp
from jax import lax
from jax.experimental import pallas as pl
from jax.experimental.pallas import tpu as pltpu, tpu_sc as plsc

NUM_CORES = 2
NUM_SUBCORES = 16
NUM_WORKERS = NUM_CORES * NUM_SUBCORES
LANE = 16
ROW_W = 128


def _build_sc_call(num_entities, num_relations, num_edges):
    assert num_edges % ROW_W == 0
    rows_total = num_edges // ROW_W
    win_rows = 16
    win_edges = win_rows * ROW_W
    full_blocks = rows_total // win_rows
    tail_rows = rows_total - full_blocks * win_rows
    per = full_blocks // NUM_WORKERS
    rem = full_blocks % NUM_WORKERS
    assert per % 4 == 0
    n_groups = per // 4
    slice_len = ((num_entities + NUM_SUBCORES - 1) // NUM_SUBCORES + LANE - 1) // LANE * LANE
    ypad = slice_len * NUM_SUBCORES

    mesh = plsc.VectorSubcoreMesh(core_axis_name="c", subcore_axis_name="s")

    @functools.partial(
        pl.kernel,
        mesh=mesh,
        out_type=jax.ShapeDtypeStruct((NUM_CORES * ypad,), jnp.float32),
        compiler_params=pltpu.CompilerParams(needs_layout_passes=False),
        scratch_types=[
            pltpu.VMEM((num_entities,), jnp.float32),
            pltpu.VMEM((ROW_W,), jnp.float32),
            pltpu.VMEM((slice_len,), jnp.float32),
            [pltpu.VMEM((2, win_edges), jnp.int32) for _ in range(4)],
            [pltpu.VMEM((win_edges,), jnp.int32) for _ in range(4)],
            [pltpu.VMEM((win_rows, ROW_W), jnp.float32) for _ in range(4)],
            pltpu.VMEM_SHARED((ypad,), jnp.float32),
            [pltpu.SemaphoreType.DMA for _ in range(4)],
            [pltpu.SemaphoreType.DMA for _ in range(4)],
            pltpu.SemaphoreType.DMA,
        ],
    )
    def sc_call(ind_hbm, rel_hbm, e_hbm, p_hbm, out_hbm,
                e_t, p_t, tmp, pair_b, rel_b, con_b, acc,
                sem_in, sem_sc, sem_tab):
        cid = lax.axis_index("c")
        sid = lax.axis_index("s")
        wid = cid * NUM_SUBCORES + sid

        start_block = wid * per + jnp.minimum(wid, rem)

        def in_start(w, r):
            eb = (start_block + w) * win_edges
            pltpu.async_copy(ind_hbm.at[pl.ds(0, 2), pl.ds(eb, win_edges)],
                             pair_b[r], sem_in[r])
            pltpu.async_copy(rel_hbm.at[pl.ds(eb, win_edges)], rel_b[r],
                             sem_in[r])

        def in_wait(w, r):
            eb = (start_block + w) * win_edges
            pltpu.make_async_copy(
                ind_hbm.at[pl.ds(0, 2), pl.ds(eb, win_edges)], pair_b[r],
                sem_in[r]).wait()
            pltpu.make_async_copy(rel_hbm.at[pl.ds(eb, win_edges)], rel_b[r],
                                  sem_in[r]).wait()

        def compute(r, nrows=win_rows):
            def _rows(j2, carry):
                for dj in range(2):
                    j = j2 * 2 + dj
                    for c in range(ROW_W // LANE):
                        off = j * ROW_W + c * LANE
                        rows = pair_b[r][0, pl.ds(off, LANE)]
                        rels = rel_b[r][pl.ds(off, LANE)]
                        ev = plsc.load_gather(e_t, [rows])
                        pv = plsc.load_gather(p_t, [rels])
                        con_b[r][j, pl.ds(c * LANE, LANE)] = ev * pv
                return carry
            lax.fori_loop(0, nrows // 2, _rows, 0)

        def scat_fire(r, nrows=win_rows):
            for j in range(nrows):
                pltpu.async_copy(con_b[r].at[j],
                                 acc.at[pair_b[r].at[1, pl.ds(j * ROW_W, ROW_W)]],
                                 sem_sc[r], add=True)

        def scat_drain(r, nrows=win_rows):
            for j in range(nrows):
                pltpu.make_async_copy(
                    con_b[r].at[j],
                    acc.at[pair_b[r].at[1, pl.ds(j * ROW_W, ROW_W)]],
                    sem_sc[r]).wait()

        pltpu.async_copy(e_hbm, e_t, sem_tab)
        pltpu.async_copy(p_hbm, p_t.at[pl.ds(0, num_relations)], sem_tab)
        in_start(0, 0)
        in_start(1, 1)

        def _zero(i, carry):
            tmp[pl.ds(i * LANE, LANE)] = jnp.zeros((LANE,), jnp.float32)
            return carry
        lax.fori_loop(0, slice_len // LANE, _zero, 0)
        pltpu.sync_copy(tmp, acc.at[pl.ds(sid * slice_len, slice_len)])
        pltpu.make_async_copy(e_hbm, e_t, sem_tab).wait()
        pltpu.make_async_copy(p_hbm, p_t.at[pl.ds(0, num_relations)],
                              sem_tab).wait()
        plsc.subcore_barrier()

        def _group(g, carry):
            for k in range(4):
                w = g * 4 + k

                @pl.when(w >= 2)
                def _():
                    scat_drain((k + 2) % 4)

                @pl.when(w + 2 < per)
                def _():
                    in_start(w + 2, (k + 2) % 4)

                in_wait(w, k)
                compute(k)
                scat_fire(k)
            return carry
        lax.fori_loop(0, n_groups, _group, 0)

        scat_drain((per - 2) % 4)
        scat_drain((per - 1) % 4)

        def tail_window(base_row, nrows):
            eb = base_row * ROW_W
            ne = nrows * ROW_W
            pltpu.sync_copy(ind_hbm.at[pl.ds(0, 2), pl.ds(eb, ne)],
                            pair_b[0].at[pl.ds(0, 2), pl.ds(0, ne)])
            pltpu.sync_copy(rel_hbm.at[pl.ds(eb, ne)],
                            rel_b[0].at[pl.ds(0, ne)])
            compute(0, nrows)
            scat_fire(0, nrows)
            scat_drain(0, nrows)

        @pl.when(wid < rem)
        def _():
            tail_window((start_block + per) * win_rows, win_rows)

        if tail_rows:
            @pl.when(wid == NUM_WORKERS - 1)
            def _():
                tail_window(full_blocks * win_rows, tail_rows)

        plsc.subcore_barrier()
        pltpu.sync_copy(acc.at[pl.ds(sid * slice_len, slice_len)], tmp)
        pltpu.sync_copy(tmp, out_hbm.at[pl.ds(cid * ypad + sid * slice_len,
                                              slice_len)])

    return sc_call


def kernel(indices, e_scores, p_scores, relation_mask):
    num_entities = e_scores.shape[0]
    num_relations = p_scores.shape[0]
    num_edges = relation_mask.shape[0]

    e_flat = e_scores.reshape(num_entities)

    sc_call = _build_sc_call(num_entities, num_relations, num_edges)
    flat = sc_call(indices, relation_mask, e_flat, p_scores)
    partials = flat.reshape(NUM_CORES, flat.shape[0] // NUM_CORES)
    return partials[0, :num_entities] + partials[1, :num_entities]


# speedup vs baseline: 1.1758x; 1.0618x over previous
import functools

import jax
import jax.numpy as jnp
from jax import lax
from jax.experimental import pallas as pl
from jax.experimental.pallas import tpu as pltpu, tpu_sc as plsc

NUM_CORES = 2
NUM_SUBCORES = 16
NUM_WORKERS = NUM_CORES * NUM_SUBCORES
LANE = 16
ROW_W = 128


def _build_sc_call(num_entities, num_relations, num_edges):
    assert num_edges % ROW_W == 0
    rows_total = num_edges // ROW_W
    win_rows = 8
    win_edges = win_rows * ROW_W
    full_blocks = rows_total // win_rows
    tail_rows = rows_total - full_blocks * win_rows
    per = full_blocks // NUM_WORKERS
    rem = full_blocks % NUM_WORKERS
    assert per % 4 == 0
    n_groups = per // 4
    slice_len = ((num_entities + NUM_SUBCORES - 1) // NUM_SUBCORES + LANE - 1) // LANE * LANE
    ypad = slice_len * NUM_SUBCORES

    mesh = plsc.VectorSubcoreMesh(core_axis_name="c", subcore_axis_name="s")

    @functools.partial(
        pl.kernel,
        mesh=mesh,
        out_type=jax.ShapeDtypeStruct((NUM_CORES * ypad,), jnp.float32),
        compiler_params=pltpu.CompilerParams(needs_layout_passes=False),
        scratch_types=[
            pltpu.VMEM((num_entities,), jnp.float32),
            pltpu.VMEM((ROW_W,), jnp.float32),
            pltpu.VMEM((slice_len,), jnp.float32),
            [pltpu.VMEM((2, win_edges), jnp.int32) for _ in range(4)],
            [pltpu.VMEM((win_edges,), jnp.int32) for _ in range(4)],
            [pltpu.VMEM((win_edges,), jnp.float32) for _ in range(4)],
            pltpu.VMEM_SHARED((ypad,), jnp.float32),
            [pltpu.SemaphoreType.DMA for _ in range(4)],
            [pltpu.SemaphoreType.DMA for _ in range(4)],
            pltpu.SemaphoreType.DMA,
        ],
    )
    def sc_call(ind_hbm, rel_hbm, e_hbm, p_hbm, out_hbm,
                e_t, p_t, tmp, pair_b, rel_b, con_b, acc,
                sem_in, sem_sc, sem_tab):
        cid = lax.axis_index("c")
        sid = lax.axis_index("s")
        wid = cid * NUM_SUBCORES + sid

        start_block = wid * per + jnp.minimum(wid, rem)

        def in_start(w, r):
            eb = (start_block + w) * win_edges
            pltpu.async_copy(ind_hbm.at[pl.ds(0, 2), pl.ds(eb, win_edges)],
                             pair_b[r], sem_in[r])
            pltpu.async_copy(rel_hbm.at[pl.ds(eb, win_edges)], rel_b[r],
                             sem_in[r])

        def in_wait(w, r):
            eb = (start_block + w) * win_edges
            pltpu.make_async_copy(
                ind_hbm.at[pl.ds(0, 2), pl.ds(eb, win_edges)], pair_b[r],
                sem_in[r]).wait()
            pltpu.make_async_copy(rel_hbm.at[pl.ds(eb, win_edges)], rel_b[r],
                                  sem_in[r]).wait()

        def compute(r, nrows=win_rows):
            for g in range(nrows * (ROW_W // LANE)):
                off = g * LANE
                rows = pair_b[r][0, pl.ds(off, LANE)]
                rels = rel_b[r][pl.ds(off, LANE)]
                ev = plsc.load_gather(e_t, [rows])
                pv = plsc.load_gather(p_t, [rels])
                con_b[r][pl.ds(off, LANE)] = ev * pv

        SCAT_W = ROW_W

        def scat_fire(r, nrows=win_rows):
            for j in range(nrows * ROW_W // SCAT_W):
                pltpu.async_copy(
                    con_b[r].at[pl.ds(j * SCAT_W, SCAT_W)],
                    acc.at[pair_b[r].at[1, pl.ds(j * SCAT_W, SCAT_W)]],
                    sem_sc[r], add=True)

        def scat_drain(r, nrows=win_rows):
            for j in range(nrows * ROW_W // SCAT_W):
                pltpu.make_async_copy(
                    con_b[r].at[pl.ds(j * SCAT_W, SCAT_W)],
                    acc.at[pair_b[r].at[1, pl.ds(j * SCAT_W, SCAT_W)]],
                    sem_sc[r]).wait()

        pltpu.async_copy(e_hbm, e_t, sem_tab)
        pltpu.async_copy(p_hbm, p_t.at[pl.ds(0, num_relations)], sem_tab)
        in_start(0, 0)
        in_start(1, 1)

        def _zero(i, carry):
            tmp[pl.ds(i * LANE, LANE)] = jnp.zeros((LANE,), jnp.float32)
            return carry
        lax.fori_loop(0, slice_len // LANE, _zero, 0)
        pltpu.sync_copy(tmp, acc.at[pl.ds(sid * slice_len, slice_len)])
        pltpu.make_async_copy(e_hbm, e_t, sem_tab).wait()
        pltpu.make_async_copy(p_hbm, p_t.at[pl.ds(0, num_relations)],
                              sem_tab).wait()
        plsc.subcore_barrier()

        def _group(g, carry):
            for k in range(4):
                w = g * 4 + k

                @pl.when(w >= 2)
                def _():
                    scat_drain((k + 2) % 4)

                @pl.when(w + 2 < per)
                def _():
                    in_start(w + 2, (k + 2) % 4)

                in_wait(w, k)
                compute(k)
                scat_fire(k)
            return carry
        lax.fori_loop(0, n_groups, _group, 0)

        scat_drain((per - 2) % 4)
        scat_drain((per - 1) % 4)

        def tail_window(base_row, nrows):
            eb = base_row * ROW_W
            ne = nrows * ROW_W
            pltpu.sync_copy(ind_hbm.at[pl.ds(0, 2), pl.ds(eb, ne)],
                            pair_b[0].at[pl.ds(0, 2), pl.ds(0, ne)])
            pltpu.sync_copy(rel_hbm.at[pl.ds(eb, ne)],
                            rel_b[0].at[pl.ds(0, ne)])
            compute(0, nrows)
            scat_fire(0, nrows)
            scat_drain(0, nrows)

        @pl.when(wid < rem)
        def _():
            tail_window((start_block + per) * win_rows, win_rows)

        if tail_rows:
            @pl.when(wid == NUM_WORKERS - 1)
            def _():
                tail_window(full_blocks * win_rows, tail_rows)

        plsc.subcore_barrier()
        pltpu.sync_copy(acc.at[pl.ds(sid * slice_len, slice_len)], tmp)
        pltpu.sync_copy(tmp, out_hbm.at[pl.ds(cid * ypad + sid * slice_len,
                                              slice_len)])

    return sc_call


def kernel(indices, e_scores, p_scores, relation_mask):
    num_entities = e_scores.shape[0]
    num_relations = p_scores.shape[0]
    num_edges = relation_mask.shape[0]

    e_flat = e_scores.reshape(num_entities)

    sc_call = _build_sc_call(num_entities, num_relations, num_edges)
    flat = sc_call(indices, relation_mask, e_flat, p_scores)
    partials = flat.reshape(NUM_CORES, flat.shape[0] // NUM_CORES)
    return partials[0, :num_entities] + partials[1, :num_entities]
